# Initial kernel scaffold; baseline (speedup 1.0000x reference)
import jax
import jax.numpy as jnp
from jax.experimental import pallas as pl


def kernel(idx, table):
    raise NotImplementedError("write your pallas kernel here")


# speedup vs baseline: 1.3397x; 1.3397x over previous
import functools

import jax
import jax.numpy as jnp
from jax import lax
from jax.experimental import pallas as pl
from jax.experimental.pallas import tpu as pltpu
from jax.experimental.pallas import tpu_sc as plsc


def _make_gather(V, D, B):
    info = plsc.get_sparse_core_info()
    NC, NS = info.num_cores, info.num_subcores
    NW = NC * NS
    assert B % NW == 0
    b_per_w = B // NW
    assert b_per_w % 8 == 0
    K = 40
    assert b_per_w % K == 0 and K % 8 == 0
    n_chunks = b_per_w // K

    mesh = plsc.VectorSubcoreMesh(core_axis_name="c", subcore_axis_name="s")

    @functools.partial(
        pl.kernel,
        mesh=mesh,
        out_type=jax.ShapeDtypeStruct((B, D), jnp.float32),
        scratch_types=[
            pltpu.VMEM((b_per_w,), jnp.int32),
            pltpu.VMEM((K, D), jnp.float32),
            pltpu.SemaphoreType.DMA,
        ],
    )
    def gather_kernel(table_hbm, idx_hbm, out_hbm, idx_v, rows_v, sem):
        wid = lax.axis_index("s") * NC + lax.axis_index("c")
        base = wid * b_per_w
        pltpu.sync_copy(idx_hbm.at[pl.ds(base, b_per_w)], idx_v)

        def body(c, carry):
            off = c * K
            pltpu.async_copy(
                table_hbm.at[idx_v.at[pl.ds(off, K)]], rows_v, sem
            ).wait()
            pltpu.sync_copy(rows_v, out_hbm.at[pl.ds(base + off, K)])
            return carry

        lax.fori_loop(0, n_chunks, body, 0)

    return gather_kernel


def kernel(idx, table):
    V, D = table.shape
    orig_shape = idx.shape
    idx_flat = idx.reshape(-1).astype(jnp.int32)
    B = idx_flat.shape[0]
    out = _make_gather(V, D, B)(table, idx_flat)
    return out.reshape(*orig_shape, D)


# speedup vs baseline: 1.4859x; 1.1091x over previous
import functools

import jax
import jax.numpy as jnp
from jax import lax
from jax.experimental import pallas as pl
from jax.experimental.pallas import tpu as pltpu
from jax.experimental.pallas import tpu_sc as plsc


def _make_gather(V, D, B):
    info = plsc.get_sparse_core_info()
    NC, NS = info.num_cores, info.num_subcores
    NW = NC * NS
    assert B % NW == 0
    b_per_w = B // NW
    assert b_per_w % 8 == 0
    K = 80
    n_full = b_per_w // K
    tail = b_per_w - n_full * K
    assert tail % 8 == 0
    n_pairs = n_full // 2
    assert n_full % 2 == 0

    mesh = plsc.VectorSubcoreMesh(core_axis_name="c", subcore_axis_name="s")

    @functools.partial(
        pl.kernel,
        mesh=mesh,
        out_type=jax.ShapeDtypeStruct((B, D), jnp.float32),
        scratch_types=[
            pltpu.VMEM((b_per_w,), jnp.int32),
            pltpu.VMEM((K, D), jnp.float32),
            pltpu.VMEM((K, D), jnp.float32),
            pltpu.SemaphoreType.DMA,
            pltpu.SemaphoreType.DMA,
            pltpu.SemaphoreType.DMA,
            pltpu.SemaphoreType.DMA,
        ],
    )
    def gather_kernel(
        table_hbm, idx_hbm, out_hbm, idx_v, buf0, buf1, g0, g1, o0, o1
    ):
        wid = lax.axis_index("s") * NC + lax.axis_index("c")
        base = wid * b_per_w
        pltpu.sync_copy(idx_hbm.at[pl.ds(base, b_per_w)], idx_v)

        def start_gather(c, buf, sem):
            pltpu.async_copy(table_hbm.at[idx_v.at[pl.ds(c * K, K)]], buf, sem)

        def wait_gather(c, buf, sem):
            pltpu.make_async_copy(
                table_hbm.at[idx_v.at[pl.ds(c * K, K)]], buf, sem
            ).wait()

        def start_out(c, buf, sem):
            pltpu.async_copy(buf, out_hbm.at[pl.ds(base + c * K, K)], sem)

        def wait_out(c, buf, sem):
            pltpu.make_async_copy(
                buf, out_hbm.at[pl.ds(base + c * K, K)], sem
            ).wait()

        start_gather(0, buf0, g0)
        start_gather(1, buf1, g1)

        def body(i, carry):
            c0 = 2 * i
            c1 = c0 + 1
            wait_gather(c0, buf0, g0)
            start_out(c0, buf0, o0)
            wait_out(c0, buf0, o0)
            start_gather(c0 + 2, buf0, g0)
            wait_gather(c1, buf1, g1)
            start_out(c1, buf1, o1)
            wait_out(c1, buf1, o1)
            start_gather(c1 + 2, buf1, g1)
            return carry

        lax.fori_loop(0, n_pairs - 1, body, 0)

        cl0 = n_full - 2
        cl1 = n_full - 1
        wait_gather(cl0, buf0, g0)
        start_out(cl0, buf0, o0)
        wait_out(cl0, buf0, o0)
        if tail:
            tb = buf0.at[pl.ds(0, tail)]
            toff = n_full * K
            pltpu.async_copy(
                table_hbm.at[idx_v.at[pl.ds(toff, tail)]], tb, g0
            )
        wait_gather(cl1, buf1, g1)
        start_out(cl1, buf1, o1)
        if tail:
            pltpu.make_async_copy(
                table_hbm.at[idx_v.at[pl.ds(toff, tail)]], tb, g0
            ).wait()
            pltpu.sync_copy(tb, out_hbm.at[pl.ds(base + toff, tail)])
        wait_out(cl1, buf1, o1)

    return gather_kernel


def kernel(idx, table):
    V, D = table.shape
    orig_shape = idx.shape
    idx_flat = idx.reshape(-1).astype(jnp.int32)
    B = idx_flat.shape[0]
    out = _make_gather(V, D, B)(table, idx_flat)
    return out.reshape(*orig_shape, D)
